# Initial kernel scaffold; baseline (speedup 1.0000x reference)
import jax
import jax.numpy as jnp
from jax.experimental import pallas as pl


def kernel(x, edge_index, edge_attr, masked_atom_mask, enc_We, enc_W1, enc_b1, enc_W2, enc_b2, e2d_W, dec_We, dec_W1, dec_b1, dec_W2, dec_b2):
    raise NotImplementedError("write your pallas kernel here")


# speedup vs baseline: 2.4794x; 2.4794x over previous
import functools

import jax
import jax.numpy as jnp
from jax import lax
from jax.experimental import pallas as pl
from jax.experimental.pallas import tpu as pltpu
from jax.experimental.pallas import tpu_sc as plsc

N = 10000
E = 320000
D = 128
DE = 16

NC = 2
NS = 16
NW = NC * NS

EW = E // NW
B = 80
NB = EW // B
N_PAD = 10240
ROWS_PER_TILE = N_PAD // NS


def _sc_msg_body(src_hbm, dst_hbm, e_hbm, x_hbm, out_hbm,
                 agg_sh, idx_s, idx_d, ev, xg, sem):
    c = lax.axis_index("c")
    s = lax.axis_index("s")
    wid = s * NC + c

    zero16 = jnp.zeros((16,), jnp.float32)

    def zrow(i, carry):
        for jj in range(D // 16):
            ev[i, pl.ds(jj * 16, 16)] = zero16
        return carry

    lax.fori_loop(0, B, zrow, 0)
    r0 = s * ROWS_PER_TILE
    for k in range(ROWS_PER_TILE // B):
        pltpu.sync_copy(ev, agg_sh.at[pl.ds(r0 + k * B, B)])
    plsc.subcore_barrier()

    base = wid * EW

    def batch(j, carry):
        off = base + j * B
        pltpu.sync_copy(src_hbm.at[pl.ds(off, B)], idx_s)
        pltpu.sync_copy(dst_hbm.at[pl.ds(off, B)], idx_d)
        pltpu.sync_copy(e_hbm.at[pl.ds(off, B)], ev)
        pltpu.async_copy(x_hbm.at[idx_s], xg, sem).wait()

        def row(i, c2):
            for jj in range(D // 16):
                sl = pl.ds(jj * 16, 16)
                ev[i, sl] = jnp.maximum(ev[i, sl] + xg[i, sl], 0.0)
            return c2

        lax.fori_loop(0, B, row, 0)
        pltpu.sync_copy(ev, agg_sh.at[idx_d], add=True)
        return carry

    lax.fori_loop(0, NB, batch, 0)
    plsc.subcore_barrier()

    pltpu.sync_copy(agg_sh.at[pl.ds(r0, ROWS_PER_TILE)],
                    out_hbm.at[c, pl.ds(r0, ROWS_PER_TILE)])


_sc_msg = pl.kernel(
    _sc_msg_body,
    out_type=jax.ShapeDtypeStruct((NC, N_PAD, D), jnp.float32),
    mesh=plsc.VectorSubcoreMesh(core_axis_name="c", subcore_axis_name="s",
                                num_cores=NC, num_subcores=NS),
    scratch_types=[
        pltpu.VMEM_SHARED((N_PAD, D), jnp.float32),
        pltpu.VMEM((B,), jnp.int32),
        pltpu.VMEM((B,), jnp.int32),
        pltpu.VMEM((B, D), jnp.float32),
        pltpu.VMEM((B, D), jnp.float32),
        pltpu.SemaphoreType.DMA,
    ],
)


_EB = 3200


def _edge_embed_body(ea_ref, we_enc_ref, we_dec_ref, out_enc_ref, out_dec_ref):
    ea = ea_ref[...]
    out_enc_ref[...] = jnp.dot(ea, we_enc_ref[...],
                               preferred_element_type=jnp.float32)
    out_dec_ref[...] = jnp.dot(ea, we_dec_ref[...],
                               preferred_element_type=jnp.float32)


def _edge_embed(edge_attr, we_enc, we_dec):
    grid = (E // _EB,)
    return pl.pallas_call(
        _edge_embed_body,
        grid=grid,
        in_specs=[
            pl.BlockSpec((_EB, DE), lambda i: (i, 0)),
            pl.BlockSpec((DE, D), lambda i: (0, 0)),
            pl.BlockSpec((DE, D), lambda i: (0, 0)),
        ],
        out_specs=[
            pl.BlockSpec((_EB, D), lambda i: (i, 0)),
            pl.BlockSpec((_EB, D), lambda i: (i, 0)),
        ],
        out_shape=[
            jax.ShapeDtypeStruct((E, D), jnp.float32),
            jax.ShapeDtypeStruct((E, D), jnp.float32),
        ],
    )(edge_attr, we_enc, we_dec)


_NBLK = 2000


def _enc_mlp_body(x_ref, a0_ref, a1_ref, m_ref, w1_ref, b1_ref, w2_ref,
                  b2_ref, e2d_ref, nr_ref, di_ref):
    h = x_ref[...] + a0_ref[0] + a1_ref[0]
    h1 = jnp.maximum(
        jnp.dot(h, w1_ref[...], preferred_element_type=jnp.float32)
        + b1_ref[...], 0.0)
    nr = jnp.dot(h1, w2_ref[...], preferred_element_type=jnp.float32) \
        + b2_ref[...]
    nr_ref[...] = nr
    di = jnp.dot(nr, e2d_ref[...], preferred_element_type=jnp.float32)
    di_ref[...] = jnp.where(m_ref[...] > 0.0, 0.0, di)


def _enc_mlp(x, agg, mask_f, w1, b1, w2, b2, e2d):
    grid = (N // _NBLK,)
    full = lambda i: (0, 0)
    blk = pl.BlockSpec((_NBLK, D), lambda i: (i, 0))
    return pl.pallas_call(
        _enc_mlp_body,
        grid=grid,
        in_specs=[
            blk,
            pl.BlockSpec((1, _NBLK, D), lambda i: (0, i, 0)),
            pl.BlockSpec((1, _NBLK, D), lambda i: (1, i, 0)),
            pl.BlockSpec((_NBLK, 1), lambda i: (i, 0)),
            pl.BlockSpec((D, D), full),
            pl.BlockSpec((1, D), full),
            pl.BlockSpec((D, D), full),
            pl.BlockSpec((1, D), full),
            pl.BlockSpec((D, D), full),
        ],
        out_specs=[blk, blk],
        out_shape=[
            jax.ShapeDtypeStruct((N, D), jnp.float32),
            jax.ShapeDtypeStruct((N, D), jnp.float32),
        ],
    )(x, agg, agg, mask_f, w1, b1, w2, b2, e2d)


def _dec_mlp_body(x_ref, a0_ref, a1_ref, w1_ref, b1_ref, w2_ref, b2_ref,
                  out_ref):
    h = x_ref[...] + a0_ref[0] + a1_ref[0]
    h1 = jnp.maximum(
        jnp.dot(h, w1_ref[...], preferred_element_type=jnp.float32)
        + b1_ref[...], 0.0)
    out_ref[...] = jnp.dot(h1, w2_ref[...],
                           preferred_element_type=jnp.float32) + b2_ref[...]


def _dec_mlp(x, agg, w1, b1, w2, b2):
    grid = (N // _NBLK,)
    full = lambda i: (0, 0)
    blk = pl.BlockSpec((_NBLK, D), lambda i: (i, 0))
    return pl.pallas_call(
        _dec_mlp_body,
        grid=grid,
        in_specs=[
            blk,
            pl.BlockSpec((1, _NBLK, D), lambda i: (0, i, 0)),
            pl.BlockSpec((1, _NBLK, D), lambda i: (1, i, 0)),
            pl.BlockSpec((D, D), full),
            pl.BlockSpec((1, D), full),
            pl.BlockSpec((D, D), full),
            pl.BlockSpec((1, D), full),
        ],
        out_specs=blk,
        out_shape=jax.ShapeDtypeStruct((N, D), jnp.float32),
    )(x, agg, agg, w1, b1, w2, b2)


def kernel(x, edge_index, edge_attr, masked_atom_mask,
           enc_We, enc_W1, enc_b1, enc_W2, enc_b2,
           e2d_W,
           dec_We, dec_W1, dec_b1, dec_W2, dec_b2):
    src = edge_index[0].astype(jnp.int32)
    dst = edge_index[1].astype(jnp.int32)
    mask_f = masked_atom_mask.astype(jnp.float32).reshape(N, 1)

    e_enc, e_dec = _edge_embed(edge_attr, enc_We, dec_We)

    agg_enc = _sc_msg(src, dst, e_enc, x)
    node_rep, decoder_input = _enc_mlp(
        x, agg_enc, mask_f,
        enc_W1, enc_b1.reshape(1, D), enc_W2, enc_b2.reshape(1, D), e2d_W)

    agg_dec = _sc_msg(src, dst, e_dec, decoder_input)
    dec_out = _dec_mlp(
        decoder_input, agg_dec,
        dec_W1, dec_b1.reshape(1, D), dec_W2, dec_b2.reshape(1, D))

    return (node_rep, dec_out)


# speedup vs baseline: 4.8554x; 1.9583x over previous
import functools

import jax
import jax.numpy as jnp
from jax import lax
from jax.experimental import pallas as pl
from jax.experimental.pallas import tpu as pltpu
from jax.experimental.pallas import tpu_sc as plsc

N = 10000
E = 320000
D = 128
DE = 16

NC = 2
NS = 16
NW = NC * NS

EW = E // NW
B = 80
NB = EW // B
N_PAD = 10112
ROWS_PER_TILE = N_PAD // NS
IR = 16


def _sc_msg_body(src_hbm, dst_hbm, e_hbm, x_hbm, out_hbm,
                 agg_sh, is_ring, ev0, ev1, xg0, xg1, idb0, idb1,
                 sem_si, sem_di, sem_e, sem_g, sem_s):
    c = lax.axis_index("c")
    s = lax.axis_index("s")
    wid = s * NC + c
    base = wid * EW

    evs = (ev0, ev1)
    xgs = (xg0, xg1)
    idbs = (idb0, idb1)

    def launch_sidx(j):
        m = lax.rem(j, IR)
        pltpu.async_copy(src_hbm.at[wid, j], is_ring.at[pl.ds(m, 1)],
                         sem_si)

    def wait_sidx(j):
        m = lax.rem(j, IR)
        pltpu.make_async_copy(src_hbm.at[wid, j],
                              is_ring.at[pl.ds(m, 1)], sem_si).wait()

    def launch_eg(j, b):
        pltpu.async_copy(e_hbm.at[pl.ds(base + j * B, B)], evs[b], sem_e)
        pltpu.async_copy(x_hbm.at[is_ring.at[lax.rem(j, IR)]], xgs[b],
                         sem_g)
        pltpu.async_copy(dst_hbm.at[wid, j], idbs[b], sem_di)

    def wait_eg(j, b):
        pltpu.make_async_copy(e_hbm.at[pl.ds(base + j * B, B)], evs[b],
                              sem_e).wait()
        pltpu.make_async_copy(x_hbm.at[is_ring.at[lax.rem(j, IR)]], xgs[b],
                              sem_g).wait()

    def compute(b):
        ev, xg = evs[b], xgs[b]

        def row(i, c2):
            for jj in range(D // 16):
                sl = pl.ds(jj * 16, 16)
                ev[i, sl] = jnp.maximum(ev[i, sl] + xg[i, sl], 0.0)
            return c2

        lax.fori_loop(0, B, row, 0)

    def scatter(j, b):
        pltpu.make_async_copy(dst_hbm.at[wid, j], idbs[b], sem_di).wait()
        pltpu.async_copy(evs[b], agg_sh.at[idbs[b].at[0]], sem_s, add=True)

    def wait_scatter(b):
        pltpu.make_async_copy(evs[b], agg_sh.at[idbs[b].at[0]], sem_s).wait()

    for jp in range(4):
        launch_sidx(jp)

    zero16 = jnp.zeros((16,), jnp.float32)

    def zrow(i, carry):
        for jj in range(D // 16):
            ev0[i, pl.ds(jj * 16, 16)] = zero16
        return carry

    lax.fori_loop(0, B, zrow, 0)
    r0 = s * ROWS_PER_TILE
    nfull, rem = divmod(ROWS_PER_TILE, B)
    for k in range(nfull):
        pltpu.sync_copy(ev0, agg_sh.at[pl.ds(r0 + k * B, B)])
    if rem:
        pltpu.sync_copy(ev0.at[pl.ds(0, rem)],
                        agg_sh.at[pl.ds(r0 + nfull * B, rem)])
    plsc.subcore_barrier()

    wait_sidx(0)
    launch_eg(0, 0)
    npairs = (NB - 1) // 2

    def pair(p, carry):
        j0 = 2 * p
        j1 = j0 + 1

        @pl.when(p > 0)
        def _():
            wait_scatter(1)

        wait_sidx(j1)
        launch_eg(j1, 1)

        @pl.when(j0 + 4 < NB)
        def _():
            launch_sidx(j0 + 4)

        @pl.when(j0 + 5 < NB)
        def _():
            launch_sidx(j0 + 5)

        wait_eg(j0, 0)
        compute(0)
        scatter(j0, 0)
        wait_scatter(0)
        wait_sidx(j0 + 2)
        launch_eg(j0 + 2, 0)
        wait_eg(j1, 1)
        compute(1)
        scatter(j1, 1)
        return carry

    lax.fori_loop(0, npairs, pair, 0)

    wait_scatter(1)
    wait_eg(NB - 1, 0)
    compute(0)
    scatter(NB - 1, 0)
    wait_scatter(0)
    plsc.subcore_barrier()

    pltpu.sync_copy(agg_sh.at[pl.ds(r0, ROWS_PER_TILE)],
                    out_hbm.at[c, pl.ds(r0, ROWS_PER_TILE)])


_sc_msg = pl.kernel(
    _sc_msg_body,
    out_type=jax.ShapeDtypeStruct((NC, N_PAD, D), jnp.float32),
    mesh=plsc.VectorSubcoreMesh(core_axis_name="c", subcore_axis_name="s",
                                num_cores=NC, num_subcores=NS),
    scratch_types=[
        pltpu.VMEM_SHARED((N_PAD, D), jnp.float32),
        pltpu.VMEM((IR, B), jnp.int32),
        pltpu.VMEM((B, D), jnp.float32),
        pltpu.VMEM((B, D), jnp.float32),
        pltpu.VMEM((B, D), jnp.float32),
        pltpu.VMEM((B, D), jnp.float32),
        pltpu.VMEM((1, B), jnp.int32),
        pltpu.VMEM((1, B), jnp.int32),
        pltpu.SemaphoreType.DMA,
        pltpu.SemaphoreType.DMA,
        pltpu.SemaphoreType.DMA,
        pltpu.SemaphoreType.DMA,
        pltpu.SemaphoreType.DMA,
    ],
)


_EB = 3200


def _edge_embed_body(ea_ref, we_enc_ref, we_dec_ref, out_enc_ref, out_dec_ref):
    ea = ea_ref[...]
    out_enc_ref[...] = jnp.dot(ea, we_enc_ref[...],
                               preferred_element_type=jnp.float32)
    out_dec_ref[...] = jnp.dot(ea, we_dec_ref[...],
                               preferred_element_type=jnp.float32)


def _edge_embed(edge_attr, we_enc, we_dec):
    grid = (E // _EB,)
    return pl.pallas_call(
        _edge_embed_body,
        grid=grid,
        in_specs=[
            pl.BlockSpec((_EB, DE), lambda i: (i, 0)),
            pl.BlockSpec((DE, D), lambda i: (0, 0)),
            pl.BlockSpec((DE, D), lambda i: (0, 0)),
        ],
        out_specs=[
            pl.BlockSpec((_EB, D), lambda i: (i, 0)),
            pl.BlockSpec((_EB, D), lambda i: (i, 0)),
        ],
        out_shape=[
            jax.ShapeDtypeStruct((E, D), jnp.float32),
            jax.ShapeDtypeStruct((E, D), jnp.float32),
        ],
    )(edge_attr, we_enc, we_dec)


_NBLK = 2000


def _enc_mlp_body(x_ref, a0_ref, a1_ref, m_ref, w1_ref, b1_ref, w2_ref,
                  b2_ref, e2d_ref, nr_ref, di_ref):
    h = x_ref[...] + a0_ref[0] + a1_ref[0]
    h1 = jnp.maximum(
        jnp.dot(h, w1_ref[...], preferred_element_type=jnp.float32)
        + b1_ref[...], 0.0)
    nr = jnp.dot(h1, w2_ref[...], preferred_element_type=jnp.float32) \
        + b2_ref[...]
    nr_ref[...] = nr
    di = jnp.dot(nr, e2d_ref[...], preferred_element_type=jnp.float32)
    di_ref[...] = jnp.where(m_ref[...] > 0.0, 0.0, di)


def _enc_mlp(x, agg, mask_f, w1, b1, w2, b2, e2d):
    grid = (N // _NBLK,)
    full = lambda i: (0, 0)
    blk = pl.BlockSpec((_NBLK, D), lambda i: (i, 0))
    return pl.pallas_call(
        _enc_mlp_body,
        grid=grid,
        in_specs=[
            blk,
            pl.BlockSpec((1, _NBLK, D), lambda i: (0, i, 0)),
            pl.BlockSpec((1, _NBLK, D), lambda i: (1, i, 0)),
            pl.BlockSpec((_NBLK, 1), lambda i: (i, 0)),
            pl.BlockSpec((D, D), full),
            pl.BlockSpec((1, D), full),
            pl.BlockSpec((D, D), full),
            pl.BlockSpec((1, D), full),
            pl.BlockSpec((D, D), full),
        ],
        out_specs=[blk, blk],
        out_shape=[
            jax.ShapeDtypeStruct((N, D), jnp.float32),
            jax.ShapeDtypeStruct((N, D), jnp.float32),
        ],
    )(x, agg, agg, mask_f, w1, b1, w2, b2, e2d)


def _dec_mlp_body(x_ref, a0_ref, a1_ref, w1_ref, b1_ref, w2_ref, b2_ref,
                  out_ref):
    h = x_ref[...] + a0_ref[0] + a1_ref[0]
    h1 = jnp.maximum(
        jnp.dot(h, w1_ref[...], preferred_element_type=jnp.float32)
        + b1_ref[...], 0.0)
    out_ref[...] = jnp.dot(h1, w2_ref[...],
                           preferred_element_type=jnp.float32) + b2_ref[...]


def _dec_mlp(x, agg, w1, b1, w2, b2):
    grid = (N // _NBLK,)
    full = lambda i: (0, 0)
    blk = pl.BlockSpec((_NBLK, D), lambda i: (i, 0))
    return pl.pallas_call(
        _dec_mlp_body,
        grid=grid,
        in_specs=[
            blk,
            pl.BlockSpec((1, _NBLK, D), lambda i: (0, i, 0)),
            pl.BlockSpec((1, _NBLK, D), lambda i: (1, i, 0)),
            pl.BlockSpec((D, D), full),
            pl.BlockSpec((1, D), full),
            pl.BlockSpec((D, D), full),
            pl.BlockSpec((1, D), full),
        ],
        out_specs=blk,
        out_shape=jax.ShapeDtypeStruct((N, D), jnp.float32),
    )(x, agg, agg, w1, b1, w2, b2)


def kernel(x, edge_index, edge_attr, masked_atom_mask,
           enc_We, enc_W1, enc_b1, enc_W2, enc_b2,
           e2d_W,
           dec_We, dec_W1, dec_b1, dec_W2, dec_b2):
    src = edge_index[0].astype(jnp.int32).reshape(NW, NB, 1, B)
    dst = edge_index[1].astype(jnp.int32).reshape(NW, NB, 1, B)
    mask_f = masked_atom_mask.astype(jnp.float32).reshape(N, 1)

    e_enc, e_dec = _edge_embed(edge_attr, enc_We, dec_We)

    agg_enc = _sc_msg(src, dst, e_enc, x)
    node_rep, decoder_input = _enc_mlp(
        x, agg_enc, mask_f,
        enc_W1, enc_b1.reshape(1, D), enc_W2, enc_b2.reshape(1, D), e2d_W)

    agg_dec = _sc_msg(src, dst, e_dec, decoder_input)
    dec_out = _dec_mlp(
        decoder_input, agg_dec,
        dec_W1, dec_b1.reshape(1, D), dec_W2, dec_b2.reshape(1, D))

    return (node_rep, dec_out)
